# baseline (speedup 1.0000x reference)
import functools

import jax
import jax.numpy as jnp
import numpy as np
from jax import lax
from jax.experimental import pallas as pl
from jax.experimental.pallas import tpu as pltpu
from jax.experimental.pallas import tpu_sc as plsc

NC, NS, L = 2, 16, 16
NW = NC * NS
NROWS, V = 64, 100000
RPW = NROWS // NW
NCHUNK = V // L
NBUCKET = 1024
CAP = 4096
T_OPT = 8.25
UNROLL = 25
GUARD = CAP - UNROLL * L
GROWS = GUARD // L
TOPBIT = np.uint32(0x80000000)
ALLBITS = np.uint32(0xFFFFFFFF)
LEVELS = ((28, 4), (24, 4), (20, 4), (16, 4), (12, 4), (8, 4), (4, 4), (0, 4))


def _mono_u32(v):
    bi = lax.bitcast_convert_type(v, jnp.int32)
    bu = lax.bitcast_convert_type(v, jnp.uint32)
    return jnp.where(bi < 0, bu ^ ALLBITS, bu ^ TOPBIT)


def _inv_mono_f32(u):
    bits = jnp.where(u >= TOPBIT, u ^ TOPBIT, u ^ ALLBITS)
    return lax.bitcast_convert_type(bits, jnp.float32)


def _pick_boundary(counts, a, kvec, lanes):
    pref = plsc.cumsum(counts)
    tot = jnp.max(pref, axis=0)
    suf = tot - pref + counts
    cnt_ge = a + suf
    nset = jnp.max(plsc.all_reduce_population_count(cnt_ge >= kvec), axis=0)
    jstar = nset - 1
    sel = lanes == jstar
    suf_j = jnp.sum(jnp.where(sel, suf, 0), axis=0)
    cnt_j = jnp.sum(jnp.where(sel, counts, 0), axis=0)
    return jstar, a + suf_j - cnt_j


def _collect(row_v, bufi_v, thresh_vec, lanes):
    guardvec = GUARD + lanes

    def body(i, carry):
        posj, idxvec = carry
        base = i * (UNROLL * L)
        posj = jnp.minimum(posj, guardvec)
        keeps = []
        for u in range(UNROLL):
            v = row_v[pl.ds(base + u * L, L)]
            keeps.append(v >= thresh_vec)
        poss = []
        for u in range(UNROLL):
            poss.append(posj)
            posj = posj + jnp.where(keeps[u], L, 0)
        for u in range(UNROLL):
            plsc.store_scatter(bufi_v, [poss[u]], idxvec + u * L,
                               mask=keeps[u])
        return posj, idxvec + UNROLL * L

    posj, _ = lax.fori_loop(0, NCHUNK // UNROLL, body, (lanes, lanes))
    return jnp.right_shift(jnp.minimum(posj, guardvec) - lanes, 4)


def _radix_select(row_v, bufi_v, h2_v, offj, kvec, lanes):
    ones = jnp.ones((L,), jnp.int32)
    zeros_i = jnp.zeros((L,), jnp.int32)
    lane_h2_base = lanes * L
    max_c = jnp.max(offj, axis=0)
    pref_u = jnp.uint32(0)
    a_run = jnp.int32(0)
    for (p, w) in LEVELS:
        for j in range(L):
            h2_v[pl.ds(j * L, L)] = zeros_i
        top_level = p + w >= 32
        hi = np.uint32(min(p + w, 31))
        pref_hi = jnp.right_shift(pref_u, hi)
        nbm1 = np.uint32((1 << w) - 1)

        def lv_body(i, _, top_level=top_level, hi=hi, pref_hi=pref_hi,
                    nbm1=nbm1, p=p):
            idx = bufi_v[pl.ds(i * L, L)]
            valid = offj > i
            v = plsc.load_gather(row_v, [idx], mask=valid)
            u = _mono_u32(v)
            if top_level:
                mk = valid
            else:
                mk = jnp.logical_and(valid, jnp.right_shift(u, hi) == pref_hi)
            sub = jnp.bitwise_and(
                jnp.right_shift(u, np.uint32(p)), nbm1).astype(jnp.int32)
            plsc.addupdate_scatter(h2_v, [lane_h2_base + sub], ones, mask=mk)
            return 0
        lax.fori_loop(0, max_c, lv_body, 0)

        counts = zeros_i
        for j in range(L):
            counts = counts + h2_v[pl.ds(j * L, L)]
        jstar, a_run = _pick_boundary(counts, a_run, kvec, lanes)
        pref_u = jnp.bitwise_or(
            pref_u, jnp.left_shift(jstar.astype(jnp.uint32), np.uint32(p)))
    return pref_u


def _zsum(row_v, bufi_v, offj, kth_vec, m):
    max_c = jnp.max(offj, axis=0)

    def body(i, acc):
        idx = bufi_v[pl.ds(i * L, L)]
        valid = offj > i
        v = plsc.load_gather(row_v, [idx], mask=valid)
        mk = jnp.logical_and(valid, v >= kth_vec)
        return acc + jnp.where(mk, jnp.exp(jnp.minimum(v - m, 0.0)), 0.0)
    zacc = lax.fori_loop(0, max_c, body, jnp.zeros((L,), jnp.float32))
    return jnp.sum(zacc, axis=0)


def _max_argmax(row_v, bufi_v, offj):
    max_c = jnp.max(offj, axis=0)

    def body(i, carry):
        mlane, idxlane = carry
        idx = bufi_v[pl.ds(i * L, L)]
        valid = offj > i
        v = plsc.load_gather(row_v, [idx], mask=valid)
        mge = jnp.logical_and(valid, v >= mlane)
        mlane = jnp.where(mge, v, mlane)
        idxlane = jnp.where(mge, idx, idxlane)
        return mlane, idxlane
    mlane, idxlane = lax.fori_loop(
        0, max_c, body,
        (jnp.full((L,), -jnp.inf, jnp.float32), jnp.zeros((L,), jnp.int32)))
    m = jnp.max(mlane, axis=0)
    x0 = jnp.max(jnp.where(mlane == m, idxlane, -1), axis=0)
    return m, x0


def _hist_threshold(row_v, hist_v, kvec, lanes):
    ones = jnp.ones((L,), jnp.int32)
    zeros_i = jnp.zeros((L,), jnp.int32)
    lane_hist_base = lanes * NBUCKET

    def zero_body(i, _):
        hist_v[pl.ds(i * L, L)] = zeros_i
        return 0
    lax.fori_loop(0, NBUCKET * L // L, zero_body, 0, unroll=8)

    def p1_body(i, _):
        v = row_v[pl.ds(i * L, L)]
        u = _mono_u32(v)
        b = jnp.right_shift(u, np.uint32(22)).astype(jnp.int32)
        plsc.addupdate_scatter(hist_v, [lane_hist_base + b], ones)
        return 0
    lax.fori_loop(0, NCHUNK, p1_body, 0, unroll=4)

    def b1_body(i, carry):
        acc, b1, found = carry
        cidx = NBUCKET // L - 1 - i
        counts = zeros_i
        for lane in range(L):
            counts = counts + hist_v[pl.ds(lane * NBUCKET + cidx * L, L)]
        jstar, _ = _pick_boundary(counts, acc, kvec, lanes)
        hit = jstar >= 0
        upd = jnp.logical_and(jnp.logical_not(found), hit)
        b1 = jnp.where(upd, cidx * L + jstar, b1)
        found = jnp.logical_or(found, hit)
        acc = acc + jnp.max(plsc.cumsum(counts), axis=0)
        return acc, b1, found
    _, b1, _ = lax.fori_loop(
        0, NBUCKET // L, b1_body,
        (jnp.int32(0), jnp.int32(0), jnp.bool_(False)))

    u_low = jnp.left_shift(b1.astype(jnp.uint32), np.uint32(22))
    return _inv_mono_f32(jnp.zeros((L,), jnp.uint32) + u_low)


def _process_row(row_v, bufi_v, hist_v, h2_v, kvec, lanes):
    t_opt_vec = jnp.full((L,), T_OPT, jnp.float32)
    offj = _collect(row_v, bufi_v, t_opt_vec, lanes)
    k_s = jnp.max(kvec, axis=0)
    fast_ok = jnp.logical_and(jnp.sum(offj, axis=0) >= k_s,
                              jnp.max(offj, axis=0) < GROWS)

    def finish(offj_f):
        m, x0 = _max_argmax(row_v, bufi_v, offj_f)
        pref_u = _radix_select(row_v, bufi_v, h2_v, offj_f, kvec, lanes)
        kth_vec = _inv_mono_f32(jnp.zeros((L,), jnp.uint32) + pref_u)
        return _zsum(row_v, bufi_v, offj_f, kth_vec, m), x0

    def fast_case():
        return finish(offj)

    def slow_case():
        t_low_vec = _hist_threshold(row_v, hist_v, kvec, lanes)
        offj2 = _collect(row_v, bufi_v, t_low_vec, lanes)
        return finish(offj2)

    return lax.cond(fast_ok, fast_case, slow_case)


def _body(logits_hbm, k_hbm, conf_out, x0_out,
          row_v, bufi_v, hist_v, h2_v, kv_v, stage_c, stage_x):
    wid = lax.axis_index("s") * NC + lax.axis_index("c")
    pltpu.sync_copy(k_hbm, kv_v)
    kvec = kv_v[...]
    lanes = lax.iota(jnp.int32, L)
    conf_acc = jnp.ones((L,), jnp.float32)
    x0_acc = jnp.zeros((L,), jnp.int32)
    for r in range(RPW):
        row = wid * RPW + r
        pltpu.sync_copy(logits_hbm.at[row], row_v)
        zsum, x0 = _process_row(row_v, bufi_v, hist_v, h2_v, kvec, lanes)
        conf_acc = jnp.where(lanes == r, zsum, conf_acc)
        x0_acc = jnp.where(lanes == r, x0, x0_acc)
    conf_acc = jnp.ones((L,), jnp.float32) / conf_acc
    stage_c[...] = conf_acc
    stage_x[...] = x0_acc
    pltpu.sync_copy(stage_c, conf_out.at[wid])
    pltpu.sync_copy(stage_x, x0_out.at[wid])


@jax.jit
def _sampler(logits, kvec):
    mesh = plsc.VectorSubcoreMesh(core_axis_name="c", subcore_axis_name="s",
                                  num_cores=NC, num_subcores=NS)
    kern = functools.partial(
        pl.kernel,
        out_type=(jax.ShapeDtypeStruct((NW, L), jnp.float32),
                  jax.ShapeDtypeStruct((NW, L), jnp.int32)),
        mesh=mesh,
        compiler_params=pltpu.CompilerParams(needs_layout_passes=False),
        scratch_types=[
            pltpu.VMEM((V,), jnp.float32),
            pltpu.VMEM((CAP,), jnp.int32),
            pltpu.VMEM((NBUCKET * L,), jnp.int32),
            pltpu.VMEM((L * L,), jnp.int32),
            pltpu.VMEM((L,), jnp.int32),
            pltpu.VMEM((L,), jnp.float32),
            pltpu.VMEM((L,), jnp.int32),
        ],
    )(_body)
    return kern(logits, kvec)


def kernel(logits, top_k):
    kvec = jnp.broadcast_to(
        jnp.minimum(jnp.asarray(top_k, jnp.int32), V), (L,))
    conf2d, x02d = _sampler(logits, kvec)
    conf = conf2d[:, :RPW].reshape(NROWS)
    x0 = x02d[:, :RPW].reshape(NROWS)
    return conf, x0, conf


# speedup vs baseline: 1.0830x; 1.0830x over previous
import functools

import jax
import jax.numpy as jnp
import numpy as np
from jax import lax
from jax.experimental import pallas as pl
from jax.experimental.pallas import tpu as pltpu
from jax.experimental.pallas import tpu_sc as plsc

NC, NS, L = 2, 16, 16
NW = NC * NS
NROWS, V = 64, 100000
RPW = NROWS // NW
NCHUNK = V // L
NBUCKET = 1024
CAP = 4096
T_OPT = 8.25
UNROLL = 10
GUARD = CAP - UNROLL * L
GROWS = GUARD // L
TOPBIT = np.uint32(0x80000000)
ALLBITS = np.uint32(0xFFFFFFFF)
LEVELS = ((28, 4), (24, 4), (20, 4), (16, 4), (12, 4), (8, 4), (4, 4), (0, 4))


def _mono_u32(v):
    bi = lax.bitcast_convert_type(v, jnp.int32)
    bu = lax.bitcast_convert_type(v, jnp.uint32)
    return jnp.where(bi < 0, bu ^ ALLBITS, bu ^ TOPBIT)


def _inv_mono_f32(u):
    bits = jnp.where(u >= TOPBIT, u ^ TOPBIT, u ^ ALLBITS)
    return lax.bitcast_convert_type(bits, jnp.float32)


def _pick_boundary(counts, a, kvec, lanes):
    pref = plsc.cumsum(counts)
    tot = jnp.max(pref, axis=0)
    suf = tot - pref + counts
    cnt_ge = a + suf
    nset = jnp.max(plsc.all_reduce_population_count(cnt_ge >= kvec), axis=0)
    jstar = nset - 1
    sel = lanes == jstar
    suf_j = jnp.sum(jnp.where(sel, suf, 0), axis=0)
    cnt_j = jnp.sum(jnp.where(sel, counts, 0), axis=0)
    return jstar, a + suf_j - cnt_j


def _collect(row_v, bufi_v, thresh_vec, lanes):
    guardvec = GUARD + lanes

    def body(i, carry):
        posj, idxvec = carry
        base = i * (UNROLL * L)
        posj = jnp.minimum(posj, guardvec)
        keeps = []
        for u in range(UNROLL):
            v = row_v[pl.ds(base + u * L, L)]
            keeps.append(v >= thresh_vec)
        poss = []
        for u in range(UNROLL):
            poss.append(posj)
            posj = posj + jnp.where(keeps[u], L, 0)
        for u in range(UNROLL):
            plsc.store_scatter(bufi_v, [poss[u]], idxvec + u * L,
                               mask=keeps[u])
        return posj, idxvec + UNROLL * L

    posj, _ = lax.fori_loop(0, NCHUNK // UNROLL, body, (lanes, lanes))
    return jnp.right_shift(jnp.minimum(posj, guardvec) - lanes, 4)


def _radix_select(row_v, bufi_v, h2_v, offj, kvec, lanes):
    ones = jnp.ones((L,), jnp.int32)
    zeros_i = jnp.zeros((L,), jnp.int32)
    lane_h2_base = lanes * L
    max_c = jnp.max(offj, axis=0)
    pref_u = jnp.uint32(0)
    a_run = jnp.int32(0)
    for (p, w) in LEVELS:
        for j in range(L):
            h2_v[pl.ds(j * L, L)] = zeros_i
        top_level = p + w >= 32
        hi = np.uint32(min(p + w, 31))
        pref_hi = jnp.right_shift(pref_u, hi)
        nbm1 = np.uint32((1 << w) - 1)

        def lv_body(i, _, top_level=top_level, hi=hi, pref_hi=pref_hi,
                    nbm1=nbm1, p=p):
            idx = bufi_v[pl.ds(i * L, L)]
            valid = offj > i
            v = plsc.load_gather(row_v, [idx], mask=valid)
            u = _mono_u32(v)
            if top_level:
                mk = valid
            else:
                mk = jnp.logical_and(valid, jnp.right_shift(u, hi) == pref_hi)
            sub = jnp.bitwise_and(
                jnp.right_shift(u, np.uint32(p)), nbm1).astype(jnp.int32)
            plsc.addupdate_scatter(h2_v, [lane_h2_base + sub], ones, mask=mk)
            return 0
        lax.fori_loop(0, max_c, lv_body, 0)

        counts = zeros_i
        for j in range(L):
            counts = counts + h2_v[pl.ds(j * L, L)]
        jstar, a_run = _pick_boundary(counts, a_run, kvec, lanes)
        pref_u = jnp.bitwise_or(
            pref_u, jnp.left_shift(jstar.astype(jnp.uint32), np.uint32(p)))
    return pref_u


def _zsum(row_v, bufi_v, offj, kth_vec, m):
    max_c = jnp.max(offj, axis=0)

    def body(i, acc):
        idx = bufi_v[pl.ds(i * L, L)]
        valid = offj > i
        v = plsc.load_gather(row_v, [idx], mask=valid)
        mk = jnp.logical_and(valid, v >= kth_vec)
        return acc + jnp.where(mk, jnp.exp(jnp.minimum(v - m, 0.0)), 0.0)
    zacc = lax.fori_loop(0, max_c, body, jnp.zeros((L,), jnp.float32))
    return jnp.sum(zacc, axis=0)


def _max_argmax(row_v, bufi_v, offj):
    max_c = jnp.max(offj, axis=0)

    def body(i, carry):
        mlane, idxlane = carry
        idx = bufi_v[pl.ds(i * L, L)]
        valid = offj > i
        v = plsc.load_gather(row_v, [idx], mask=valid)
        mge = jnp.logical_and(valid, v >= mlane)
        mlane = jnp.where(mge, v, mlane)
        idxlane = jnp.where(mge, idx, idxlane)
        return mlane, idxlane
    mlane, idxlane = lax.fori_loop(
        0, max_c, body,
        (jnp.full((L,), -jnp.inf, jnp.float32), jnp.zeros((L,), jnp.int32)))
    m = jnp.max(mlane, axis=0)
    x0 = jnp.max(jnp.where(mlane == m, idxlane, -1), axis=0)
    return m, x0


def _hist_threshold(row_v, hist_v, kvec, lanes):
    ones = jnp.ones((L,), jnp.int32)
    zeros_i = jnp.zeros((L,), jnp.int32)
    lane_hist_base = lanes * NBUCKET

    def zero_body(i, _):
        hist_v[pl.ds(i * L, L)] = zeros_i
        return 0
    lax.fori_loop(0, NBUCKET * L // L, zero_body, 0, unroll=8)

    def p1_body(i, _):
        v = row_v[pl.ds(i * L, L)]
        u = _mono_u32(v)
        b = jnp.right_shift(u, np.uint32(22)).astype(jnp.int32)
        plsc.addupdate_scatter(hist_v, [lane_hist_base + b], ones)
        return 0
    lax.fori_loop(0, NCHUNK, p1_body, 0, unroll=4)

    def b1_body(i, carry):
        acc, b1, found = carry
        cidx = NBUCKET // L - 1 - i
        counts = zeros_i
        for lane in range(L):
            counts = counts + hist_v[pl.ds(lane * NBUCKET + cidx * L, L)]
        jstar, _ = _pick_boundary(counts, acc, kvec, lanes)
        hit = jstar >= 0
        upd = jnp.logical_and(jnp.logical_not(found), hit)
        b1 = jnp.where(upd, cidx * L + jstar, b1)
        found = jnp.logical_or(found, hit)
        acc = acc + jnp.max(plsc.cumsum(counts), axis=0)
        return acc, b1, found
    _, b1, _ = lax.fori_loop(
        0, NBUCKET // L, b1_body,
        (jnp.int32(0), jnp.int32(0), jnp.bool_(False)))

    u_low = jnp.left_shift(b1.astype(jnp.uint32), np.uint32(22))
    return _inv_mono_f32(jnp.zeros((L,), jnp.uint32) + u_low)


def _process_row(row_v, bufi_v, hist_v, h2_v, kvec, lanes):
    t_opt_vec = jnp.full((L,), T_OPT, jnp.float32)
    offj = _collect(row_v, bufi_v, t_opt_vec, lanes)
    k_s = jnp.max(kvec, axis=0)
    fast_ok = jnp.logical_and(jnp.sum(offj, axis=0) >= k_s,
                              jnp.max(offj, axis=0) < GROWS)

    def finish(offj_f):
        m, x0 = _max_argmax(row_v, bufi_v, offj_f)
        pref_u = _radix_select(row_v, bufi_v, h2_v, offj_f, kvec, lanes)
        kth_vec = _inv_mono_f32(jnp.zeros((L,), jnp.uint32) + pref_u)
        return _zsum(row_v, bufi_v, offj_f, kth_vec, m), x0

    def fast_case():
        return finish(offj)

    def slow_case():
        t_low_vec = _hist_threshold(row_v, hist_v, kvec, lanes)
        offj2 = _collect(row_v, bufi_v, t_low_vec, lanes)
        return finish(offj2)

    return lax.cond(fast_ok, fast_case, slow_case)


def _body(logits_hbm, k_hbm, conf_out, x0_out,
          row_v, bufi_v, hist_v, h2_v, kv_v, stage_c, stage_x):
    wid = lax.axis_index("s") * NC + lax.axis_index("c")
    pltpu.sync_copy(k_hbm, kv_v)
    kvec = kv_v[...]
    lanes = lax.iota(jnp.int32, L)
    conf_acc = jnp.ones((L,), jnp.float32)
    x0_acc = jnp.zeros((L,), jnp.int32)
    for r in range(RPW):
        row = wid * RPW + r
        pltpu.sync_copy(logits_hbm.at[row], row_v)
        zsum, x0 = _process_row(row_v, bufi_v, hist_v, h2_v, kvec, lanes)
        conf_acc = jnp.where(lanes == r, zsum, conf_acc)
        x0_acc = jnp.where(lanes == r, x0, x0_acc)
    conf_acc = jnp.ones((L,), jnp.float32) / conf_acc
    stage_c[...] = conf_acc
    stage_x[...] = x0_acc
    pltpu.sync_copy(stage_c, conf_out.at[wid])
    pltpu.sync_copy(stage_x, x0_out.at[wid])


@jax.jit
def _sampler(logits, kvec):
    mesh = plsc.VectorSubcoreMesh(core_axis_name="c", subcore_axis_name="s",
                                  num_cores=NC, num_subcores=NS)
    kern = functools.partial(
        pl.kernel,
        out_type=(jax.ShapeDtypeStruct((NW, L), jnp.float32),
                  jax.ShapeDtypeStruct((NW, L), jnp.int32)),
        mesh=mesh,
        compiler_params=pltpu.CompilerParams(needs_layout_passes=False),
        scratch_types=[
            pltpu.VMEM((V,), jnp.float32),
            pltpu.VMEM((CAP,), jnp.int32),
            pltpu.VMEM((NBUCKET * L,), jnp.int32),
            pltpu.VMEM((L * L,), jnp.int32),
            pltpu.VMEM((L,), jnp.int32),
            pltpu.VMEM((L,), jnp.float32),
            pltpu.VMEM((L,), jnp.int32),
        ],
    )(_body)
    return kern(logits, kvec)


def kernel(logits, top_k):
    kvec = jnp.broadcast_to(
        jnp.minimum(jnp.asarray(top_k, jnp.int32), V), (L,))
    conf2d, x02d = _sampler(logits, kvec)
    conf = conf2d[:, :RPW].reshape(NROWS)
    x0 = x02d[:, :RPW].reshape(NROWS)
    return conf, x0, conf
